# baseline (speedup 1.0000x reference)
import functools

import jax
import jax.numpy as jnp
from jax import lax
from jax.experimental import pallas as pl
from jax.experimental.pallas import tpu as pltpu
from jax.experimental.pallas import tpu_sc as plsc

B_ROWS = 4096
SEQ = 200
EMBED = 64

_info = plsc.get_sparse_core_info()
NC = _info.num_cores
NS = _info.num_subcores
NW = NC * NS

TOTAL = B_ROWS * SEQ
PER_W = TOTAL // NW
CHUNK = 128
STEPS = PER_W // CHUNK


def _sc_gather(table, idx3):
    mesh = plsc.VectorSubcoreMesh(core_axis_name="c", subcore_axis_name="s")

    @functools.partial(
        pl.kernel,
        mesh=mesh,
        out_type=jax.ShapeDtypeStruct((TOTAL, EMBED), jnp.float32),
        scratch_types=[
            pltpu.VMEM((STEPS, CHUNK), jnp.int32),
            pltpu.VMEM((CHUNK, EMBED), jnp.float32),
            pltpu.SemaphoreType.DMA,
        ],
        compiler_params=pltpu.CompilerParams(use_tc_tiling_on_sc=False),
    )
    def k(table_hbm, idx_hbm, out_hbm, idx_v, buf, gsem):
        wid = lax.axis_index("s") * NC + lax.axis_index("c")
        pltpu.sync_copy(idx_hbm.at[wid], idx_v)

        def step(j, carry):
            pltpu.async_copy(table_hbm.at[idx_v.at[j]], buf, gsem).wait()
            pltpu.sync_copy(buf, out_hbm.at[pl.ds((wid * STEPS + j) * CHUNK, CHUNK)])
            return carry

        lax.fori_loop(0, STEPS, step, 0)

    return k(table, idx3)


def kernel(input, table):
    idx3 = input.reshape(NW, STEPS, CHUNK)
    out = _sc_gather(table, idx3)
    return out.reshape(B_ROWS, SEQ, EMBED)


# speedup vs baseline: 1.1192x; 1.1192x over previous
import functools

import jax
import jax.numpy as jnp
from jax import lax
from jax.experimental import pallas as pl
from jax.experimental.pallas import tpu as pltpu
from jax.experimental.pallas import tpu_sc as plsc

B_ROWS = 4096
SEQ = 200
EMBED = 64

_info = plsc.get_sparse_core_info()
NC = _info.num_cores
NS = _info.num_subcores
NW = NC * NS

TOTAL = B_ROWS * SEQ
PER_W = TOTAL // NW
CHUNK = 128
STEPS = PER_W // CHUNK
N_BUF = 4
GROUPS = STEPS // N_BUF


def _sc_gather(table, idx3):
    mesh = plsc.VectorSubcoreMesh(core_axis_name="c", subcore_axis_name="s")

    @functools.partial(
        pl.kernel,
        mesh=mesh,
        out_type=jax.ShapeDtypeStruct((TOTAL, EMBED), jnp.float32),
        scratch_types=[
            pltpu.VMEM((STEPS, CHUNK), jnp.int32),
            *[pltpu.VMEM((CHUNK, EMBED), jnp.float32) for _ in range(N_BUF)],
            *[pltpu.SemaphoreType.DMA for _ in range(N_BUF)],
        ],
        compiler_params=pltpu.CompilerParams(use_tc_tiling_on_sc=False),
    )
    def k(table_hbm, idx_hbm, out_hbm, idx_v, *rest):
        bufs = rest[:N_BUF]
        gsems = rest[N_BUF:]
        wid = lax.axis_index("s") * NC + lax.axis_index("c")
        pltpu.sync_copy(idx_hbm.at[wid], idx_v)
        base = wid * STEPS

        def start_gather(j, b):
            pltpu.async_copy(table_hbm.at[idx_v.at[j]], bufs[b], gsems[b])

        def wait_gather(j, b):
            pltpu.make_async_copy(table_hbm.at[idx_v.at[j]], bufs[b], gsems[b]).wait()

        def copy_out(j, b):
            pltpu.sync_copy(bufs[b], out_hbm.at[pl.ds((base + j) * CHUNK, CHUNK)])

        for b in range(N_BUF):
            start_gather(b, b)

        def body(i, carry):
            for b in range(N_BUF):
                j = i * N_BUF + b
                wait_gather(j, b)
                copy_out(j, b)
                start_gather(j + N_BUF, b)
            return carry

        lax.fori_loop(0, GROUPS - 1, body, 0)

        for b in range(N_BUF):
            j = (GROUPS - 1) * N_BUF + b
            wait_gather(j, b)
            copy_out(j, b)

    return k(table, idx3)


def kernel(input, table):
    idx3 = input.reshape(NW, STEPS, CHUNK)
    out = _sc_gather(table, idx3)
    return out.reshape(B_ROWS, SEQ, EMBED)
